# Initial kernel scaffold; baseline (speedup 1.0000x reference)
import jax
import jax.numpy as jnp
from jax.experimental import pallas as pl


def kernel(pos, normal, batch, W1a, b1a, W1b, b1b, W2a, b2a, W2b, b2b, W3a, b3a, W3b, b3b):
    raise NotImplementedError("write your pallas kernel here")


# speedup vs baseline: 3.9074x; 3.9074x over previous
import functools

import jax
import jax.numpy as jnp
from jax import lax
from jax.experimental import pallas as pl
from jax.experimental.pallas import tpu as pltpu
from jax.experimental.pallas import tpu_sc as plsc

N = 8192
K = 16
ROWS = 256
NODE_BLK = 512


def _knn_body(pos_blk_ref, posT_ref, idx_ref):
    pb = pos_blk_ref[:]
    pT = posT_ref[:]
    sq_all = jnp.sum(pT * pT, axis=0, keepdims=True)
    sq_b = jnp.sum(pb * pb, axis=1, keepdims=True)
    d2 = sq_b + sq_all - 2.0 * jnp.dot(pb, pT, preferred_element_type=jnp.float32)

    cols = lax.broadcasted_iota(jnp.int32, (ROWS, N), 1)
    lane_k = lax.broadcasted_iota(jnp.int32, (ROWS, K), 1)
    idx_acc = jnp.zeros((ROWS, K), jnp.int32)

    def body(k, carry):
        d2, idx_acc = carry
        v = jnp.min(d2, axis=1, keepdims=True)
        am = jnp.min(jnp.where(d2 <= v, cols, jnp.int32(N)), axis=1)
        d2 = jnp.where(cols == am[:, None], jnp.float32(3e38), d2)
        idx_acc = jnp.where(lane_k == k, am[:, None], idx_acc)
        return d2, idx_acc

    _, idx_acc = lax.fori_loop(0, K, body, (d2, idx_acc))
    idx_ref[:] = idx_acc


def _knn(pos_pad):
    posT = pos_pad.T
    return pl.pallas_call(
        _knn_body,
        grid=(N // ROWS,),
        in_specs=[
            pl.BlockSpec((ROWS, 8), lambda i: (i, 0)),
            pl.BlockSpec((8, N), lambda i: (0, 0)),
        ],
        out_specs=pl.BlockSpec((ROWS, K), lambda i: (i, 0)),
        out_shape=jax.ShapeDtypeStruct((N, K), jnp.int32),
    )(pos_pad, posT)


def _pre_body(x_ref, pos_ref, wx_ref, wr_ref, ba_ref, a_ref, c_ref):
    p = jnp.dot(pos_ref[:], wr_ref[:], preferred_element_type=jnp.float32)
    a_ref[:] = jnp.dot(x_ref[:], wx_ref[:], preferred_element_type=jnp.float32) + p
    c_ref[:] = ba_ref[:] - p


def _pre(x, pos_pad, wx, wr, ba):
    do = wx.shape[1]
    return pl.pallas_call(
        _pre_body,
        out_shape=(
            jax.ShapeDtypeStruct((N, do), jnp.float32),
            jax.ShapeDtypeStruct((N, do), jnp.float32),
        ),
    )(x, pos_pad, wx, wr, ba)


NW = 32
NBUF = 4


def _sc_gather(table, idx2d):
    do = table.shape[1]
    nchunks = idx2d.shape[0]
    cpw = nchunks // NW

    mesh = plsc.VectorSubcoreMesh(core_axis_name="c", subcore_axis_name="s")

    @functools.partial(
        pl.kernel,
        mesh=mesh,
        out_type=jax.ShapeDtypeStruct((N * K, do), jnp.float32),
        scratch_types=[
            pltpu.VMEM((cpw, 128), jnp.int32),
            pltpu.VMEM((NBUF, 128, do), jnp.float32),
        ] + [pltpu.SemaphoreType.DMA] * NBUF,
    )
    def k(table_hbm, idx_hbm, out_hbm, idx_v, rows_v, *sems):
        wid = lax.axis_index("s") * 2 + lax.axis_index("c")
        base = wid * cpw
        pltpu.sync_copy(idx_hbm.at[pl.ds(base, cpw)], idx_v)

        def outer(g, carry):
            handles = []
            for b in range(NBUF):
                ch = g * NBUF + b
                handles.append(pltpu.async_copy(
                    table_hbm.at[idx_v.at[ch]], rows_v.at[b], sems[b]))
            for b in range(NBUF):
                handles[b].wait()
                row0 = (base + g * NBUF + b) * 128
                pltpu.sync_copy(rows_v.at[b], out_hbm.at[pl.ds(row0, 128)])
            return carry

        lax.fori_loop(0, cpw // NBUF, outer, 0)

    return k(table, idx2d)


def _post_body(g_ref, c_ref, wb_ref, bb_ref, out_ref):
    do = g_ref.shape[1]
    do2 = wb_ref.shape[1]
    g = g_ref[:]
    c = c_ref[:]
    m = jnp.maximum(g.reshape(NODE_BLK, K, do) + c[:, None, :], 0.0)
    h = jnp.dot(m.reshape(NODE_BLK * K, do), wb_ref[:],
                preferred_element_type=jnp.float32) + bb_ref[:]
    out_ref[:] = jnp.maximum(jnp.max(h.reshape(NODE_BLK, K, do2), axis=1), 0.0)


def _post(gathered, c, wb, bb):
    do = gathered.shape[1]
    do2 = wb.shape[1]
    nblk = N // NODE_BLK
    return pl.pallas_call(
        _post_body,
        grid=(nblk,),
        in_specs=[
            pl.BlockSpec((NODE_BLK * K, do), lambda i: (i, 0)),
            pl.BlockSpec((NODE_BLK, do), lambda i: (i, 0)),
            pl.BlockSpec((do, do2), lambda i: (0, 0)),
            pl.BlockSpec((1, do2), lambda i: (0, 0)),
        ],
        out_specs=pl.BlockSpec((NODE_BLK, do2), lambda i: (i, 0)),
        out_shape=jax.ShapeDtypeStruct((N, do2), jnp.float32),
    )(gathered, c, wb, bb)


def _pad_to(m, rows=None, cols=None):
    r = m.shape[0] if rows is None else rows
    c = m.shape[1] if cols is None else cols
    out = jnp.zeros((r, c), m.dtype)
    return out.at[: m.shape[0], : m.shape[1]].set(m)


def _layer(x, pos_pad, wa, ba, wb, bb):
    dx = wa.shape[0] - 3
    dxp = 8 if dx < 8 else dx
    wx = _pad_to(wa[:dx], rows=dxp, cols=128)
    if x.shape[1] != dxp:
        x = _pad_to(x, cols=dxp)
    wr = _pad_to(wa[dx:], rows=8, cols=128)
    ba_p = _pad_to(ba.reshape(1, -1), cols=128)
    a, c = _pre(x, pos_pad, wx, wr, ba_p)
    return a, c


def kernel(pos, normal, batch, W1a, b1a, W1b, b1b, W2a, b2a, W2b, b2b,
           W3a, b3a, W3b, b3b):
    pos_pad = jnp.concatenate([pos, jnp.zeros((N, 5), pos.dtype)], axis=1)

    idx = _knn(pos_pad)
    idx2d = idx.reshape(N * K // 128, 128)

    x1 = jnp.concatenate([pos, normal], axis=1)

    outs = []
    x = x1
    for wa, ba, wb, bb in ((W1a, b1a, W1b, b1b),
                           (W2a, b2a, W2b, b2b),
                           (W3a, b3a, W3b, b3b)):
        a, c = _layer(x, pos_pad, wa, ba, wb, bb)
        g = _sc_gather(a, idx2d)
        wb_p = _pad_to(wb, rows=128)
        x = _post(g, c, wb_p, bb.reshape(1, -1))
        outs.append(x)
    return tuple(outs)


# speedup vs baseline: 3.9336x; 1.0067x over previous
import functools

import jax
import jax.numpy as jnp
from jax import lax
from jax.experimental import pallas as pl
from jax.experimental.pallas import tpu as pltpu
from jax.experimental.pallas import tpu_sc as plsc

N = 8192
K = 16
ROWS = 256
NODE_BLK = 512


G = N // 128
INF = 3e38


def _knn_body(pos_blk_ref, pt_ref, idx_ref, d2_ref):
    pb = pos_blk_ref[:]
    sq_b = jnp.sum(pb * pb, axis=1, keepdims=True)
    lane = lax.broadcasted_iota(jnp.int32, (ROWS, 128), 1)
    lane_k = lax.broadcasted_iota(jnp.int32, (ROWS, K), 1)
    acc0 = jnp.full((ROWS, 128), INF, jnp.float32)
    accg0 = jnp.zeros((ROWS, 128), jnp.int32)
    bigi = jnp.int32(1 << 30)

    def extract(acc, accg):
        v = jnp.min(acc, axis=1, keepdims=True)
        col = accg * 128 + lane
        return jnp.min(jnp.where(acc <= v, col, bigi), axis=1)

    def g0(g, carry):
        acc, accg = carry
        pt = pt_ref[g]
        sqg = jnp.sum(pt * pt, axis=0, keepdims=True)
        dv = sq_b + sqg - 2.0 * jnp.dot(pb, pt, preferred_element_type=jnp.float32)
        d2_ref[g] = dv
        c = dv < acc
        return jnp.where(c, dv, acc), jnp.where(c, g, accg)

    acc, accg = lax.fori_loop(0, G, g0, (acc0, accg0))
    am = extract(acc, accg)
    idx_acc = jnp.where(lane_k == 0, am[:, None], 0)

    def round_k(k, carry):
        am_prev, idx_acc = carry

        def gb(g, c2):
            acc, accg, tgt = c2
            dv = d2_ref[g]
            dv = jnp.where(lane == tgt, INF, dv)
            d2_ref[g] = dv
            c = dv < acc
            return (jnp.where(c, dv, acc), jnp.where(c, g, accg), tgt - 128)

        acc, accg, _ = lax.fori_loop(
            0, G, gb, (acc0, accg0, am_prev[:, None]))
        am = extract(acc, accg)
        return am, jnp.where(lane_k == k, am[:, None], idx_acc)

    _, idx_acc = lax.fori_loop(1, K, round_k, (am, idx_acc))
    idx_ref[:] = idx_acc


def _knn(pos_pad):
    pt = jnp.transpose(pos_pad.reshape(G, 128, 8), (0, 2, 1))
    return pl.pallas_call(
        _knn_body,
        grid=(N // ROWS,),
        in_specs=[
            pl.BlockSpec((ROWS, 8), lambda i: (i, 0)),
            pl.BlockSpec((G, 8, 128), lambda i: (0, 0, 0)),
        ],
        out_specs=pl.BlockSpec((ROWS, K), lambda i: (i, 0)),
        out_shape=jax.ShapeDtypeStruct((N, K), jnp.int32),
        scratch_shapes=[pltpu.VMEM((G, ROWS, 128), jnp.float32)],
    )(pos_pad, pt)


def _pre_body(x_ref, pos_ref, wx_ref, wr_ref, ba_ref, a_ref, c_ref):
    p = jnp.dot(pos_ref[:], wr_ref[:], preferred_element_type=jnp.float32)
    a_ref[:] = jnp.dot(x_ref[:], wx_ref[:], preferred_element_type=jnp.float32) + p
    c_ref[:] = ba_ref[:] - p


def _pre(x, pos_pad, wx, wr, ba):
    do = wx.shape[1]
    return pl.pallas_call(
        _pre_body,
        out_shape=(
            jax.ShapeDtypeStruct((N, do), jnp.float32),
            jax.ShapeDtypeStruct((N, do), jnp.float32),
        ),
    )(x, pos_pad, wx, wr, ba)


NW = 32
NBUF = 4


def _sc_gather(table, idx2d):
    do = table.shape[1]
    nchunks = idx2d.shape[0]
    cpw = nchunks // NW

    mesh = plsc.VectorSubcoreMesh(core_axis_name="c", subcore_axis_name="s")

    @functools.partial(
        pl.kernel,
        mesh=mesh,
        out_type=jax.ShapeDtypeStruct((N * K, do), jnp.float32),
        scratch_types=[
            pltpu.VMEM((cpw, 128), jnp.int32),
            pltpu.VMEM((NBUF, 128, do), jnp.float32),
        ] + [pltpu.SemaphoreType.DMA] * NBUF,
    )
    def k(table_hbm, idx_hbm, out_hbm, idx_v, rows_v, *sems):
        wid = lax.axis_index("s") * 2 + lax.axis_index("c")
        base = wid * cpw
        pltpu.sync_copy(idx_hbm.at[pl.ds(base, cpw)], idx_v)

        def outer(g, carry):
            handles = []
            for b in range(NBUF):
                ch = g * NBUF + b
                handles.append(pltpu.async_copy(
                    table_hbm.at[idx_v.at[ch]], rows_v.at[b], sems[b]))
            for b in range(NBUF):
                handles[b].wait()
                row0 = (base + g * NBUF + b) * 128
                pltpu.sync_copy(rows_v.at[b], out_hbm.at[pl.ds(row0, 128)])
            return carry

        lax.fori_loop(0, cpw // NBUF, outer, 0)

    return k(table, idx2d)


def _post_body(g_ref, c_ref, wb_ref, bb_ref, out_ref):
    do = g_ref.shape[1]
    do2 = wb_ref.shape[1]
    g = g_ref[:]
    c = c_ref[:]
    m = jnp.maximum(g.reshape(NODE_BLK, K, do) + c[:, None, :], 0.0)
    h = jnp.dot(m.reshape(NODE_BLK * K, do), wb_ref[:],
                preferred_element_type=jnp.float32) + bb_ref[:]
    out_ref[:] = jnp.maximum(jnp.max(h.reshape(NODE_BLK, K, do2), axis=1), 0.0)


def _post(gathered, c, wb, bb):
    do = gathered.shape[1]
    do2 = wb.shape[1]
    nblk = N // NODE_BLK
    return pl.pallas_call(
        _post_body,
        grid=(nblk,),
        in_specs=[
            pl.BlockSpec((NODE_BLK * K, do), lambda i: (i, 0)),
            pl.BlockSpec((NODE_BLK, do), lambda i: (i, 0)),
            pl.BlockSpec((do, do2), lambda i: (0, 0)),
            pl.BlockSpec((1, do2), lambda i: (0, 0)),
        ],
        out_specs=pl.BlockSpec((NODE_BLK, do2), lambda i: (i, 0)),
        out_shape=jax.ShapeDtypeStruct((N, do2), jnp.float32),
    )(gathered, c, wb, bb)


def _pad_to(m, rows=None, cols=None):
    r = m.shape[0] if rows is None else rows
    c = m.shape[1] if cols is None else cols
    out = jnp.zeros((r, c), m.dtype)
    return out.at[: m.shape[0], : m.shape[1]].set(m)


def _layer(x, pos_pad, wa, ba, wb, bb):
    dx = wa.shape[0] - 3
    dxp = 8 if dx < 8 else dx
    wx = _pad_to(wa[:dx], rows=dxp, cols=128)
    if x.shape[1] != dxp:
        x = _pad_to(x, cols=dxp)
    wr = _pad_to(wa[dx:], rows=8, cols=128)
    ba_p = _pad_to(ba.reshape(1, -1), cols=128)
    a, c = _pre(x, pos_pad, wx, wr, ba_p)
    return a, c


def kernel(pos, normal, batch, W1a, b1a, W1b, b1b, W2a, b2a, W2b, b2b,
           W3a, b3a, W3b, b3b):
    pos_pad = jnp.concatenate([pos, jnp.zeros((N, 5), pos.dtype)], axis=1)

    idx = _knn(pos_pad)
    idx2d = idx.reshape(N * K // 128, 128)

    x1 = jnp.concatenate([pos, normal], axis=1)

    outs = []
    x = x1
    for wa, ba, wb, bb in ((W1a, b1a, W1b, b1b),
                           (W2a, b2a, W2b, b2b),
                           (W3a, b3a, W3b, b3b)):
        a, c = _layer(x, pos_pad, wa, ba, wb, bb)
        g = _sc_gather(a, idx2d)
        wb_p = _pad_to(wb, rows=128)
        x = _post(g, c, wb_p, bb.reshape(1, -1))
        outs.append(x)
    return tuple(outs)


# speedup vs baseline: 6.1293x; 1.5582x over previous
import functools

import jax
import jax.numpy as jnp
from jax import lax
from jax.experimental import pallas as pl
from jax.experimental.pallas import tpu as pltpu
from jax.experimental.pallas import tpu_sc as plsc

N = 8192
K = 16
ROWS = 256
NODE_BLK = 512


G = N // 128
INF = 3e38


def _knn_body(pos_blk_ref, pt_ref, idx_ref, d2_ref):
    pb = pos_blk_ref[:]
    sq_b = jnp.sum(pb * pb, axis=1, keepdims=True)
    lane = lax.broadcasted_iota(jnp.int32, (ROWS, 128), 1)
    lane_k = lax.broadcasted_iota(jnp.int32, (ROWS, K), 1)
    acc0 = jnp.full((ROWS, 128), INF, jnp.float32)
    accg0 = jnp.zeros((ROWS, 128), jnp.int32)
    bigi = jnp.int32(1 << 30)

    def extract(acc, accg):
        v = jnp.min(acc, axis=1, keepdims=True)
        col = accg * 128 + lane
        return jnp.min(jnp.where(acc <= v, col, bigi), axis=1)

    U = 8

    def g0(gu, carry):
        acc, accg = carry
        for u in range(U):
            g = gu * U + u
            pt = pt_ref[g]
            sqg = jnp.sum(pt * pt, axis=0, keepdims=True)
            dv = sq_b + sqg - 2.0 * jnp.dot(
                pb, pt, preferred_element_type=jnp.float32)
            d2_ref[g] = dv
            c = dv < acc
            acc = jnp.where(c, dv, acc)
            accg = jnp.where(c, g, accg)
        return acc, accg

    acc, accg = lax.fori_loop(0, G // U, g0, (acc0, accg0))
    am = extract(acc, accg)
    idx_acc = jnp.where(lane_k == 0, am[:, None], 0)

    def round_k(k, carry):
        am_prev, idx_acc = carry

        def gb(gu, c2):
            acc, accg, tgt = c2
            for u in range(U):
                g = gu * U + u
                dv = d2_ref[g]
                dv = jnp.where(lane == tgt, INF, dv)
                d2_ref[g] = dv
                c = dv < acc
                acc = jnp.where(c, dv, acc)
                accg = jnp.where(c, g, accg)
                tgt = tgt - 128
            return acc, accg, tgt

        acc, accg, _ = lax.fori_loop(
            0, G // U, gb, (acc0, accg0, am_prev[:, None]))
        am = extract(acc, accg)
        return am, jnp.where(lane_k == k, am[:, None], idx_acc)

    _, idx_acc = lax.fori_loop(1, K, round_k, (am, idx_acc))
    idx_ref[:] = idx_acc


def _knn(pos_pad):
    pt = jnp.transpose(pos_pad.reshape(G, 128, 8), (0, 2, 1))
    return pl.pallas_call(
        _knn_body,
        grid=(N // ROWS,),
        in_specs=[
            pl.BlockSpec((ROWS, 8), lambda i: (i, 0)),
            pl.BlockSpec((G, 8, 128), lambda i: (0, 0, 0)),
        ],
        out_specs=pl.BlockSpec((ROWS, K), lambda i: (i, 0)),
        out_shape=jax.ShapeDtypeStruct((N, K), jnp.int32),
        scratch_shapes=[pltpu.VMEM((G, ROWS, 128), jnp.float32)],
    )(pos_pad, pt)


def _pre_body(x_ref, pos_ref, wx_ref, wr_ref, ba_ref, a_ref, c_ref):
    p = jnp.dot(pos_ref[:], wr_ref[:], preferred_element_type=jnp.float32)
    a_ref[:] = jnp.dot(x_ref[:], wx_ref[:], preferred_element_type=jnp.float32) + p
    c_ref[:] = ba_ref[:] - p


def _pre(x, pos_pad, wx, wr, ba):
    do = wx.shape[1]
    return pl.pallas_call(
        _pre_body,
        out_shape=(
            jax.ShapeDtypeStruct((N, do), jnp.float32),
            jax.ShapeDtypeStruct((N, do), jnp.float32),
        ),
    )(x, pos_pad, wx, wr, ba)


NW = 32
NBUF = 4


def _sc_gather(table, idx2d):
    do = table.shape[1]
    nchunks = idx2d.shape[0]
    cpw = nchunks // NW

    mesh = plsc.VectorSubcoreMesh(core_axis_name="c", subcore_axis_name="s")

    @functools.partial(
        pl.kernel,
        mesh=mesh,
        out_type=jax.ShapeDtypeStruct((N * K, do), jnp.float32),
        scratch_types=[
            pltpu.VMEM((cpw, 128), jnp.int32),
            pltpu.VMEM((NBUF, 128, do), jnp.float32),
        ] + [pltpu.SemaphoreType.DMA] * NBUF,
    )
    def k(table_hbm, idx_hbm, out_hbm, idx_v, rows_v, *sems):
        wid = lax.axis_index("s") * 2 + lax.axis_index("c")
        base = wid * cpw
        pltpu.sync_copy(idx_hbm.at[pl.ds(base, cpw)], idx_v)

        def outer(g, carry):
            handles = []
            for b in range(NBUF):
                ch = g * NBUF + b
                handles.append(pltpu.async_copy(
                    table_hbm.at[idx_v.at[ch]], rows_v.at[b], sems[b]))
            for b in range(NBUF):
                handles[b].wait()
                row0 = (base + g * NBUF + b) * 128
                pltpu.sync_copy(rows_v.at[b], out_hbm.at[pl.ds(row0, 128)])
            return carry

        lax.fori_loop(0, cpw // NBUF, outer, 0)

    return k(table, idx2d)


def _post_body(g_ref, c_ref, wb_ref, bb_ref, out_ref):
    do = g_ref.shape[1]
    do2 = wb_ref.shape[1]
    g = g_ref[:]
    c = c_ref[:]
    m = jnp.maximum(g.reshape(NODE_BLK, K, do) + c[:, None, :], 0.0)
    h = jnp.dot(m.reshape(NODE_BLK * K, do), wb_ref[:],
                preferred_element_type=jnp.float32) + bb_ref[:]
    out_ref[:] = jnp.maximum(jnp.max(h.reshape(NODE_BLK, K, do2), axis=1), 0.0)


def _post(gathered, c, wb, bb):
    do = gathered.shape[1]
    do2 = wb.shape[1]
    nblk = N // NODE_BLK
    return pl.pallas_call(
        _post_body,
        grid=(nblk,),
        in_specs=[
            pl.BlockSpec((NODE_BLK * K, do), lambda i: (i, 0)),
            pl.BlockSpec((NODE_BLK, do), lambda i: (i, 0)),
            pl.BlockSpec((do, do2), lambda i: (0, 0)),
            pl.BlockSpec((1, do2), lambda i: (0, 0)),
        ],
        out_specs=pl.BlockSpec((NODE_BLK, do2), lambda i: (i, 0)),
        out_shape=jax.ShapeDtypeStruct((N, do2), jnp.float32),
    )(gathered, c, wb, bb)


def _pad_to(m, rows=None, cols=None):
    r = m.shape[0] if rows is None else rows
    c = m.shape[1] if cols is None else cols
    out = jnp.zeros((r, c), m.dtype)
    return out.at[: m.shape[0], : m.shape[1]].set(m)


def _layer(x, pos_pad, wa, ba, wb, bb):
    dx = wa.shape[0] - 3
    dxp = 8 if dx < 8 else dx
    wx = _pad_to(wa[:dx], rows=dxp, cols=128)
    if x.shape[1] != dxp:
        x = _pad_to(x, cols=dxp)
    wr = _pad_to(wa[dx:], rows=8, cols=128)
    ba_p = _pad_to(ba.reshape(1, -1), cols=128)
    a, c = _pre(x, pos_pad, wx, wr, ba_p)
    return a, c


def kernel(pos, normal, batch, W1a, b1a, W1b, b1b, W2a, b2a, W2b, b2b,
           W3a, b3a, W3b, b3b):
    pos_pad = jnp.concatenate([pos, jnp.zeros((N, 5), pos.dtype)], axis=1)

    idx = _knn(pos_pad)
    idx2d = idx.reshape(N * K // 128, 128)

    x1 = jnp.concatenate([pos, normal], axis=1)

    outs = []
    x = x1
    for wa, ba, wb, bb in ((W1a, b1a, W1b, b1b),
                           (W2a, b2a, W2b, b2b),
                           (W3a, b3a, W3b, b3b)):
        a, c = _layer(x, pos_pad, wa, ba, wb, bb)
        g = _sc_gather(a, idx2d)
        wb_p = _pad_to(wb, rows=128)
        x = _post(g, c, wb_p, bb.reshape(1, -1))
        outs.append(x)
    return tuple(outs)
